# baseline (speedup 1.0000x reference)
import functools

import jax
import jax.numpy as jnp
from jax.experimental import pallas as pl
from jax.experimental.pallas import tpu as pltpu


def _vq_body(z_ref, cbn_ref, cb_ref, csq_ref, q_ref, loss_ref, *, n_groups):
    n = pl.program_id(0)
    g = pl.program_id(1)
    t = pl.program_id(2)

    z = z_ref[0]
    cbn = cbn_ref[0]
    cb = cb_ref[0]
    csq = csq_ref[0]

    mm = jax.lax.dot_general(cbn, z, (((1,), (0,)), ((), ())),
                             preferred_element_type=jnp.float32)
    dist = mm + csq

    idx = jnp.argmin(dist, axis=0)
    onehot = (jax.lax.broadcasted_iota(jnp.int32, dist.shape, 0)
              == idx[None, :]).astype(jnp.float32)
    q = jax.lax.dot_general(cb, onehot, (((0,), (0,)), ((), ())),
                            preferred_element_type=jnp.float32)
    q_ref[0] = q

    @pl.when((n == 0) & (g == 0) & (t == 0))
    def _init():
        loss_ref[0, 0] = 0.0

    @pl.when(g == n_groups - 1)
    def _acc():
        r = z - q
        loss_ref[0, 0] += jnp.sum(r * r)


def kernel(z, codebooks):
    N, C, T0 = z.shape
    G, CB, K = codebooks.shape
    T = (C * T0) // (G * K)
    zr = jnp.reshape(z, (N, G * K, T))

    cb_neg2 = -2.0 * codebooks
    csq = jnp.sum(codebooks * codebooks, axis=2, keepdims=True)

    TB = 4096 if T % 4096 == 0 else T

    q, loss_sum = pl.pallas_call(
        functools.partial(_vq_body, n_groups=G),
        grid=(N, G, T // TB),
        in_specs=[
            pl.BlockSpec((1, K, TB), lambda n, g, t: (n, g, t)),
            pl.BlockSpec((1, CB, K), lambda n, g, t: (g, 0, 0)),
            pl.BlockSpec((1, CB, K), lambda n, g, t: (g, 0, 0)),
            pl.BlockSpec((1, CB, 1), lambda n, g, t: (g, 0, 0)),
        ],
        out_specs=[
            pl.BlockSpec((1, K, TB), lambda n, g, t: (n, g, t)),
            pl.BlockSpec(memory_space=pltpu.SMEM),
        ],
        out_shape=[
            jax.ShapeDtypeStruct((N, G * K, T), jnp.float32),
            jax.ShapeDtypeStruct((1, 1), jnp.float32),
        ],
    )(zr, cb_neg2, codebooks, csq)

    vq_loss = loss_sum[0, 0] * (0.25 / (N * K * T))
    return jnp.reshape(q, (N, C, T0)), vq_loss


# speedup vs baseline: 1.1483x; 1.1483x over previous
import functools

import jax
import jax.numpy as jnp
from jax.experimental import pallas as pl
from jax.experimental.pallas import tpu as pltpu


def _vq_body(z_ref, cbn_ref, cb_ref, csq_ref, q_ref, loss_ref, *,
             n_groups, k_dim):
    n = pl.program_id(0)
    rows = z_ref.shape[1] // n_groups
    t = z_ref.shape[2] * rows // k_dim

    @pl.when(n == 0)
    def _init():
        loss_ref[0, 0] = 0.0

    outs = []
    for g in range(n_groups):
        zb = z_ref[0, rows * g:rows * (g + 1), :].reshape(k_dim, t)
        cbn = cbn_ref[g]
        mm = jax.lax.dot_general(cbn, zb, (((1,), (0,)), ((), ())),
                                 preferred_element_type=jnp.float32)
        dist = mm + csq_ref[g]
        idx = jnp.argmin(dist, axis=0)
        onehot = (jax.lax.broadcasted_iota(jnp.int32, dist.shape, 0)
                  == idx[None, :]).astype(jnp.float32)
        q = jax.lax.dot_general(cb_ref[g], onehot, (((0,), (0,)), ((), ())),
                                preferred_element_type=jnp.float32)
        outs.append(q.reshape(rows, z_ref.shape[2]))

        if g == n_groups - 1:
            r = zb - q
            loss_ref[0, 0] += jnp.sum(r * r)

    q_ref[0] = jnp.concatenate(outs, axis=0)


def kernel(z, codebooks):
    N, C, T0 = z.shape
    G, CB, K = codebooks.shape
    T = (C * T0) // (G * K)

    cb_neg2 = -2.0 * codebooks
    csq = jnp.sum(codebooks * codebooks, axis=2, keepdims=True)

    q, loss_sum = pl.pallas_call(
        functools.partial(_vq_body, n_groups=G, k_dim=K),
        grid=(N,),
        in_specs=[
            pl.BlockSpec((1, C, T0), lambda n: (n, 0, 0)),
            pl.BlockSpec((G, CB, K), lambda n: (0, 0, 0)),
            pl.BlockSpec((G, CB, K), lambda n: (0, 0, 0)),
            pl.BlockSpec((G, CB, 1), lambda n: (0, 0, 0)),
        ],
        out_specs=[
            pl.BlockSpec((1, C, T0), lambda n: (n, 0, 0)),
            pl.BlockSpec(memory_space=pltpu.SMEM),
        ],
        out_shape=[
            jax.ShapeDtypeStruct((N, C, T0), jnp.float32),
            jax.ShapeDtypeStruct((1, 1), jnp.float32),
        ],
    )(z, cb_neg2, codebooks, csq)

    vq_loss = loss_sum[0, 0] * (0.25 / (N * K * T))
    return q, vq_loss
